# baseline (speedup 1.0000x reference)
import functools

import jax
import jax.numpy as jnp
from jax import lax
from jax.experimental import pallas as pl
from jax.experimental.pallas import tpu as pltpu
from jax.experimental.pallas import tpu_sc as plsc

_R = 1024


def _rsqrt(x):
    i = lax.bitcast_convert_type(x, jnp.int32)
    i = jnp.int32(0x5F3759DF) - lax.shift_right_arithmetic(i, 1)
    y = lax.bitcast_convert_type(i, jnp.float32)
    for _ in range(3):
        y = y * (1.5 - 0.5 * x * y * y)
    return y


def _sc_body(n, xyz, scal, rot, op, dc, rest, out,
             out_v, xyz_v, scal_v, rot_v, op_v, rest_v, dc0_v, dc1_v, dc2_v, sem):
    nfull = n // _R
    nw = 32
    iters = (nfull + nw - 1) // nw
    wid = lax.axis_index("s") * 2 + lax.axis_index("c")

    def process(base, size):
        cols = pl.ds(base, size)
        vcols = pl.ds(0, size)
        copies = [
            pltpu.async_copy(xyz.at[:, cols], xyz_v.at[:, vcols], sem),
            pltpu.async_copy(scal.at[:, cols], scal_v.at[:, vcols], sem),
            pltpu.async_copy(rot.at[:, cols], rot_v.at[:, vcols], sem),
            pltpu.async_copy(op.at[:, cols], op_v.at[:, vcols], sem),
            pltpu.async_copy(dc.at[0, 0, cols], dc0_v.at[vcols], sem),
            pltpu.async_copy(dc.at[1, 0, cols], dc1_v.at[vcols], sem),
            pltpu.async_copy(dc.at[2, 0, cols], dc2_v.at[vcols], sem),
        ]
        for c in copies:
            c.wait()

        def act_body(j, carry2):
            sl = pl.ds(j * 16, 16)
            for r in range(3):
                out_v[r, sl] = xyz_v[r, sl]
            for r in range(3):
                out_v[3 + r, sl] = jnp.exp(scal_v[r, sl])
            q = [rot_v[r, sl] for r in range(4)]
            n2 = q[0] * q[0] + q[1] * q[1] + q[2] * q[2] + q[3] * q[3]
            inv = jnp.where(n2 >= 1e-24, _rsqrt(n2), 1e12)
            for r in range(4):
                out_v[6 + r, sl] = q[r] * inv
            out_v[10, sl] = 1.0 / (1.0 + jnp.exp(-op_v[0, sl]))
            out_v[11, sl] = dc0_v[sl]
            out_v[12, sl] = dc1_v[sl]
            out_v[13, sl] = dc2_v[sl]
            return carry2

        lax.fori_loop(0, size // 16, act_body, 0)

        for k in range(3):
            pltpu.async_copy(rest.at[k, :, cols], rest_v.at[:, vcols], sem).wait()

            def rest_copy(j, carry2, _k=k):
                sl = pl.ds(j * 16, 16)
                for i in range(15):
                    out_v[14 + 3 * i + _k, sl] = rest_v[i, sl]
                return carry2

            lax.fori_loop(0, size // 16, rest_copy, 0)

        pltpu.async_copy(out_v.at[:, vcols], out.at[:, cols], sem).wait()

    def chunk_body(it, carry):
        ck = wid + it * nw

        @pl.when(ck < nfull)
        def _():
            process(ck * _R, _R)

        return carry

    lax.fori_loop(0, iters, chunk_body, 0)


def _tc_tail_body(xyz_ref, scal_ref, rot_ref, op_ref, dc_ref, rest_ref, _, out_ref):
    out_ref[0:3, :] = xyz_ref[...]
    out_ref[3:6, :] = jnp.exp(scal_ref[...])
    q = rot_ref[...]
    norm = jnp.sqrt(jnp.sum(q * q, axis=0, keepdims=True))
    out_ref[6:10, :] = q / jnp.maximum(norm, 1e-12)
    out_ref[10:11, :] = 1.0 / (1.0 + jnp.exp(-op_ref[...]))
    out_ref[11:14, :] = dc_ref[:, 0, :]
    for i in range(15):
        out_ref[14 + 3 * i : 17 + 3 * i, :] = rest_ref[:, i, :]


def kernel(xyz, features_dc, features_rest, scaling, rotation, opacity):
    n = xyz.shape[0]
    xyz_t = xyz.T
    scal_t = scaling.T
    rot_t = rotation.T
    op_t = opacity.T
    dc_t = features_dc.transpose(2, 1, 0)
    rest_t = features_rest.transpose(2, 1, 0)

    mesh = plsc.VectorSubcoreMesh(core_axis_name="c", subcore_axis_name="s")
    sc_out = pl.kernel(
        functools.partial(_sc_body, n),
        out_type=jax.ShapeDtypeStruct((59, n), jnp.float32),
        mesh=mesh,
        scratch_types=[
            pltpu.VMEM((59, _R), jnp.float32),
            pltpu.VMEM((3, _R), jnp.float32),
            pltpu.VMEM((3, _R), jnp.float32),
            pltpu.VMEM((4, _R), jnp.float32),
            pltpu.VMEM((1, _R), jnp.float32),
            pltpu.VMEM((15, _R), jnp.float32),
            pltpu.VMEM((_R,), jnp.float32),
            pltpu.VMEM((_R,), jnp.float32),
            pltpu.VMEM((_R,), jnp.float32),
            pltpu.SemaphoreType.DMA,
        ],
    )(xyz_t, scal_t, rot_t, op_t, dc_t, rest_t)

    tb = (n // _R) * _R // _R

    def rows2(c):
        return pl.BlockSpec((c, _R), lambda i: (0, tb))

    def rows3(c, m):
        return pl.BlockSpec((c, m, _R), lambda i: (0, 0, tb))

    out = pl.pallas_call(
        _tc_tail_body,
        grid=(1,),
        in_specs=[
            rows2(3), rows2(3), rows2(4), rows2(1), rows3(3, 1), rows3(3, 15),
            pl.BlockSpec(memory_space=pl.ANY),
        ],
        out_specs=rows2(59),
        out_shape=jax.ShapeDtypeStruct((59, n), jnp.float32),
        input_output_aliases={6: 0},
    )(xyz_t, scal_t, rot_t, op_t, dc_t, rest_t, sc_out)
    return out.T


# speedup vs baseline: 3.6544x; 3.6544x over previous
import jax
import jax.numpy as jnp
from jax.experimental import pallas as pl

_W = 32768


def _fuse_body(xyz_ref, scal_ref, rot_ref, op_ref, dc_ref, rest_ref, out_ref):
    out_ref[0:3, :] = xyz_ref[...]
    out_ref[3:6, :] = jnp.exp(scal_ref[...])
    q = rot_ref[...]
    norm = jnp.sqrt(jnp.sum(q * q, axis=0, keepdims=True))
    out_ref[6:10, :] = q / jnp.maximum(norm, 1e-12)
    out_ref[10:11, :] = 1.0 / (1.0 + jnp.exp(-op_ref[...]))
    out_ref[11:14, :] = dc_ref[:, 0, :]
    for i in range(15):
        out_ref[14 + 3 * i : 17 + 3 * i, :] = rest_ref[:, i, :]


def kernel(xyz, features_dc, features_rest, scaling, rotation, opacity):
    n = xyz.shape[0]
    xyz_t = xyz.T
    scal_t = scaling.T
    rot_t = rotation.T
    op_t = opacity.T
    dc_t = features_dc.transpose(2, 1, 0)
    rest_t = features_rest.transpose(2, 1, 0)

    grid = pl.cdiv(n, _W)

    def rows2(c):
        return pl.BlockSpec((c, _W), lambda i: (0, i))

    def rows3(c, m):
        return pl.BlockSpec((c, m, _W), lambda i: (0, 0, i))

    out = pl.pallas_call(
        _fuse_body,
        grid=(grid,),
        in_specs=[rows2(3), rows2(3), rows2(4), rows2(1), rows3(3, 1), rows3(3, 15)],
        out_specs=rows2(59),
        out_shape=jax.ShapeDtypeStruct((59, n), jnp.float32),
    )(xyz_t, scal_t, rot_t, op_t, dc_t, rest_t)
    return out.T


# speedup vs baseline: 3.6688x; 1.0039x over previous
import jax
import jax.numpy as jnp
from jax.experimental import pallas as pl

_W = 40960


def _fuse_body(xyz_ref, scal_ref, rot_ref, op_ref, dc_ref, rest_ref, out_ref):
    out_ref[0:3, :] = xyz_ref[...]
    out_ref[3:6, :] = jnp.exp(scal_ref[...])
    q = rot_ref[...]
    norm = jnp.sqrt(jnp.sum(q * q, axis=0, keepdims=True))
    out_ref[6:10, :] = q / jnp.maximum(norm, 1e-12)
    out_ref[10:11, :] = 1.0 / (1.0 + jnp.exp(-op_ref[...]))
    out_ref[11:14, :] = dc_ref[:, 0, :]
    for i in range(15):
        out_ref[14 + 3 * i : 17 + 3 * i, :] = rest_ref[:, i, :]


def kernel(xyz, features_dc, features_rest, scaling, rotation, opacity):
    n = xyz.shape[0]
    xyz_t = xyz.T
    scal_t = scaling.T
    rot_t = rotation.T
    op_t = opacity.T
    dc_t = features_dc.transpose(2, 1, 0)
    rest_t = features_rest.transpose(2, 1, 0)

    grid = pl.cdiv(n, _W)

    def rows2(c):
        return pl.BlockSpec((c, _W), lambda i: (0, i))

    def rows3(c, m):
        return pl.BlockSpec((c, m, _W), lambda i: (0, 0, i))

    out = pl.pallas_call(
        _fuse_body,
        grid=(grid,),
        in_specs=[rows2(3), rows2(3), rows2(4), rows2(1), rows3(3, 1), rows3(3, 15)],
        out_specs=rows2(59),
        out_shape=jax.ShapeDtypeStruct((59, n), jnp.float32),
    )(xyz_t, scal_t, rot_t, op_t, dc_t, rest_t)
    return out.T


# speedup vs baseline: 3.6813x; 1.0034x over previous
import jax
import jax.numpy as jnp
from jax.experimental import pallas as pl

_W = 51200


def _fuse_body(xyz_ref, scal_ref, rot_ref, op_ref, dc_ref, rest_ref, out_ref):
    out_ref[0:3, :] = xyz_ref[...]
    out_ref[3:6, :] = jnp.exp(scal_ref[...])
    q = rot_ref[...]
    norm = jnp.sqrt(jnp.sum(q * q, axis=0, keepdims=True))
    out_ref[6:10, :] = q / jnp.maximum(norm, 1e-12)
    out_ref[10:11, :] = 1.0 / (1.0 + jnp.exp(-op_ref[...]))
    out_ref[11:14, :] = dc_ref[:, 0, :]
    for i in range(15):
        out_ref[14 + 3 * i : 17 + 3 * i, :] = rest_ref[:, i, :]


def kernel(xyz, features_dc, features_rest, scaling, rotation, opacity):
    n = xyz.shape[0]
    xyz_t = xyz.T
    scal_t = scaling.T
    rot_t = rotation.T
    op_t = opacity.T
    dc_t = features_dc.transpose(2, 1, 0)
    rest_t = features_rest.transpose(2, 1, 0)

    grid = pl.cdiv(n, _W)

    def rows2(c):
        return pl.BlockSpec((c, _W), lambda i: (0, i))

    def rows3(c, m):
        return pl.BlockSpec((c, m, _W), lambda i: (0, 0, i))

    out = pl.pallas_call(
        _fuse_body,
        grid=(grid,),
        in_specs=[rows2(3), rows2(3), rows2(4), rows2(1), rows3(3, 1), rows3(3, 15)],
        out_specs=rows2(59),
        out_shape=jax.ShapeDtypeStruct((59, n), jnp.float32),
    )(xyz_t, scal_t, rot_t, op_t, dc_t, rest_t)
    return out.T
